# baseline (speedup 1.0000x reference)
import functools

import jax
import jax.numpy as jnp
from jax import lax
from jax.experimental import pallas as pl
from jax.experimental.pallas import tpu as pltpu
from jax.experimental.pallas import tpu_sc as plsc

N_NODES = 10000
N_PAD = 10240
D = 128
EPS = 1e-5

NC = 2
NS = 16
NW = NC * NS

E_CHUNK = 80


def _sc_segment_sum(x, eflat, n_chunks, per_worker):
    rows_per_tile = N_PAD // NS
    E = per_worker * NW

    mesh = plsc.VectorSubcoreMesh(core_axis_name="c", subcore_axis_name="s")

    @functools.partial(
        pl.kernel,
        out_type=jax.ShapeDtypeStruct((NC, N_NODES, D), jnp.float32),
        mesh=mesh,
        scratch_types=[
            pltpu.VMEM((per_worker,), jnp.int32),
            pltpu.VMEM((E_CHUNK,), jnp.int32),
            pltpu.VMEM((E_CHUNK,), jnp.int32),
            pltpu.VMEM((E_CHUNK, D), jnp.float32),
            pltpu.VMEM((E_CHUNK, D), jnp.float32),
            pltpu.VMEM_SHARED((N_PAD, D), jnp.float32),
            pltpu.SemaphoreType.DMA,
            pltpu.SemaphoreType.DMA,
            pltpu.SemaphoreType.DMA,
            pltpu.SemaphoreType.DMA,
            pltpu.SemaphoreType.DMA,
            pltpu.SemaphoreType.DMA,
        ],
    )
    def k(x_hbm, e_hbm, out_hbm, sidx, didx_a, didx_b,
          rows_a, rows_b, agg, sem_a, sem_b, dsem_a, dsem_b,
          ssem_a, ssem_b):
        cid = lax.axis_index("c")
        sid = lax.axis_index("s")
        wid = cid * NS + sid
        src0 = wid * per_worker
        dst0 = E + wid * per_worker

        zero16 = jnp.zeros((16,), jnp.float32)

        def zfill(r, carry):
            for j in range(D // 16):
                rows_a[r, pl.ds(j * 16, 16)] = zero16
            return carry

        lax.fori_loop(0, E_CHUNK, zfill, 0)
        row0 = sid * rows_per_tile
        for j in range(rows_per_tile // E_CHUNK):
            pltpu.sync_copy(rows_a, agg.at[pl.ds(row0 + j * E_CHUNK, E_CHUNK)])

        pltpu.sync_copy(e_hbm.at[pl.ds(src0, per_worker)], sidx)

        plsc.subcore_barrier()

        NSTR = 5
        QE = E_CHUNK // NSTR

        def start_gather(i, nrows, nsem):
            for q in range(NSTR):
                pltpu.async_copy(
                    x_hbm.at[sidx.at[pl.ds(i * E_CHUNK + q * QE, QE)]],
                    nrows.at[pl.ds(q * QE, QE)], nsem)

        pltpu.async_copy(e_hbm.at[pl.ds(dst0, E_CHUNK)], didx_a, dsem_a)
        start_gather(0, rows_a, sem_a)

        def step(i, rows, sem, didx, dsem, ssem,
                 nrows, nsem, ndidx, ndsem, nssem):
            @pl.when(i + 1 < n_chunks)
            def _():
                @pl.when(i >= 1)
                def _():
                    pltpu.make_async_copy(nrows, agg.at[ndidx], nssem).wait()

                pltpu.async_copy(
                    e_hbm.at[pl.ds(dst0 + (i + 1) * E_CHUNK, E_CHUNK)],
                    ndidx, ndsem)
                start_gather(i + 1, nrows, nsem)

            pltpu.make_async_copy(
                x_hbm.at[sidx.at[pl.ds(i * E_CHUNK, E_CHUNK)]],
                rows, sem).wait()
            pltpu.make_async_copy(
                e_hbm.at[pl.ds(dst0 + i * E_CHUNK, E_CHUNK)],
                didx, dsem).wait()
            pltpu.async_copy(rows, agg.at[didx], ssem, add=True)

        def body(i, carry):
            @pl.when(lax.rem(i, 2) == 0)
            def _():
                step(i, rows_a, sem_a, didx_a, dsem_a, ssem_a,
                     rows_b, sem_b, didx_b, dsem_b, ssem_b)

            @pl.when(lax.rem(i, 2) == 1)
            def _():
                step(i, rows_b, sem_b, didx_b, dsem_b, ssem_b,
                     rows_a, sem_a, didx_a, dsem_a, ssem_a)

            return carry

        lax.fori_loop(0, n_chunks, body, 0)
        pltpu.make_async_copy(rows_a, agg.at[didx_a], ssem_a).wait()
        pltpu.make_async_copy(rows_b, agg.at[didx_b], ssem_b).wait()

        plsc.subcore_barrier()

        last = N_NODES - (NS - 1) * rows_per_tile

        @pl.when(sid < NS - 1)
        def _():
            pltpu.sync_copy(agg.at[pl.ds(row0, rows_per_tile)],
                            out_hbm.at[cid, pl.ds(row0, rows_per_tile)])

        @pl.when(sid == NS - 1)
        def _():
            pltpu.sync_copy(agg.at[pl.ds((NS - 1) * rows_per_tile, last)],
                            out_hbm.at[cid, pl.ds((NS - 1) * rows_per_tile, last)])

    return k(x, eflat)


def _tc_root(x, W_root, b2):
    BLK = 1000
    n_blk = N_NODES // BLK

    def body(x_ref, wt_ref, b_ref, o_ref):
        o_ref[...] = lax.dot_general(
            x_ref[...], wt_ref[...], (((1,), (1,)), ((), ())),
            preferred_element_type=jnp.float32) + b_ref[...]

    return pl.pallas_call(
        body,
        grid=(n_blk,),
        in_specs=[
            pl.BlockSpec((BLK, D), lambda i: (i, 0)),
            pl.BlockSpec((D, D), lambda i: (0, 0)),
            pl.BlockSpec((1, D), lambda i: (0, 0)),
        ],
        out_specs=pl.BlockSpec((BLK, D), lambda i: (i, 0)),
        out_shape=jax.ShapeDtypeStruct((N_NODES, D), jnp.float32),
    )(x, W_root, b2)


def _tc_finish(partials, xw, W_rel, g2, be2):
    BLK = 1000
    n_blk = N_NODES // BLK

    def body(p_ref, xw_ref, wr_ref, g_ref, be_ref, o_ref,
             h_scr, s_scr, s2_scr):
        ph = pl.program_id(0)
        i = pl.program_id(1)

        @pl.when(ph == 0)
        def _():
            a = p_ref[0] + p_ref[1]
            h = lax.dot_general(a, wr_ref[...], (((1,), (1,)), ((), ())),
                                preferred_element_type=jnp.float32)
            h = h + xw_ref[...]
            h_scr[pl.ds(i * BLK, BLK), :] = h
            o_ref[...] = h

            @pl.when(i == 0)
            def _():
                s_scr[...] = jnp.zeros((1, D), jnp.float32)
                s2_scr[...] = jnp.zeros((1, D), jnp.float32)

            s_scr[...] += jnp.sum(h, axis=0, keepdims=True)
            s2_scr[...] += jnp.sum(h * h, axis=0, keepdims=True)

        @pl.when(ph == 1)
        def _():
            mean = s_scr[...] * (1.0 / N_NODES)
            var = s2_scr[...] * (1.0 / N_NODES) - mean * mean
            scale = g_ref[...] * lax.rsqrt(var + EPS)
            shift = be_ref[...] - mean * scale
            h = h_scr[pl.ds(i * BLK, BLK), :]
            o_ref[...] = jnp.maximum(h * scale + shift, 0.0)

    return pl.pallas_call(
        body,
        grid=(2, n_blk),
        in_specs=[
            pl.BlockSpec((NC, BLK, D), lambda ph, i: (0, (1 - ph) * i, 0)),
            pl.BlockSpec((BLK, D), lambda ph, i: ((1 - ph) * i, 0)),
            pl.BlockSpec((D, D), lambda ph, i: (0, 0)),
            pl.BlockSpec((1, D), lambda ph, i: (0, 0)),
            pl.BlockSpec((1, D), lambda ph, i: (0, 0)),
        ],
        out_specs=pl.BlockSpec((BLK, D), lambda ph, i: (i, 0)),
        scratch_shapes=[
            pltpu.VMEM((N_NODES, D), jnp.float32),
            pltpu.VMEM((1, D), jnp.float32),
            pltpu.VMEM((1, D), jnp.float32),
        ],
        out_shape=jax.ShapeDtypeStruct((N_NODES, D), jnp.float32),
    )(partials, xw, W_rel, g2, be2)


def kernel(x, edge_index, batch, W_rel, W_root, b_rel, gamma, beta):
    del batch
    ei = edge_index.astype(jnp.int32)
    E = ei.shape[1]
    per_worker = E // NW
    n_chunks = per_worker // E_CHUNK
    eflat = ei.reshape(2 * E)
    partials = _sc_segment_sum(x, eflat, n_chunks, per_worker)
    xw = _tc_root(x, W_root, b_rel.reshape(1, D))
    return _tc_finish(partials, xw, W_rel, gamma.reshape(1, D),
                      beta.reshape(1, D))


# speedup vs baseline: 1.0126x; 1.0126x over previous
import functools

import jax
import jax.numpy as jnp
from jax import lax
from jax.experimental import pallas as pl
from jax.experimental.pallas import tpu as pltpu
from jax.experimental.pallas import tpu_sc as plsc

N_NODES = 10000
N_PAD = 10240
D = 128
EPS = 1e-5

NC = 2
NS = 16
NW = NC * NS

E_CHUNK = 80


def _sc_segment_sum(x, eflat, n_chunks, per_worker):
    rows_per_tile = N_PAD // NS
    E = per_worker * NW

    mesh = plsc.VectorSubcoreMesh(core_axis_name="c", subcore_axis_name="s")

    @functools.partial(
        pl.kernel,
        out_type=jax.ShapeDtypeStruct((NC, N_NODES, D), jnp.float32),
        mesh=mesh,
        scratch_types=[
            pltpu.VMEM((per_worker,), jnp.int32),
            pltpu.VMEM((E_CHUNK,), jnp.int32),
            pltpu.VMEM((E_CHUNK,), jnp.int32),
            pltpu.VMEM((E_CHUNK, D), jnp.float32),
            pltpu.VMEM((E_CHUNK, D), jnp.float32),
            pltpu.VMEM_SHARED((N_PAD, D), jnp.float32),
            pltpu.SemaphoreType.DMA,
            pltpu.SemaphoreType.DMA,
            pltpu.SemaphoreType.DMA,
            pltpu.SemaphoreType.DMA,
            pltpu.SemaphoreType.DMA,
            pltpu.SemaphoreType.DMA,
        ],
    )
    def k(x_hbm, e_hbm, out_hbm, sidx, didx_a, didx_b,
          rows_a, rows_b, agg, sem_a, sem_b, dsem_a, dsem_b,
          ssem_a, ssem_b):
        cid = lax.axis_index("c")
        sid = lax.axis_index("s")
        wid = cid * NS + sid
        src0 = wid * per_worker
        dst0 = E + wid * per_worker

        pltpu.async_copy(e_hbm.at[pl.ds(src0, per_worker)], sidx, sem_b)

        zero16 = jnp.zeros((16,), jnp.float32)

        def zfill(r, carry):
            for j in range(D // 16):
                rows_a[r, pl.ds(j * 16, 16)] = zero16
            return carry

        lax.fori_loop(0, E_CHUNK, zfill, 0)
        row0 = sid * rows_per_tile
        for j in range(rows_per_tile // E_CHUNK):
            pltpu.async_copy(rows_a,
                             agg.at[pl.ds(row0 + j * E_CHUNK, E_CHUNK)],
                             sem_a)
        for j in range(rows_per_tile // E_CHUNK):
            pltpu.make_async_copy(
                rows_a, agg.at[pl.ds(row0 + j * E_CHUNK, E_CHUNK)],
                sem_a).wait()
        pltpu.make_async_copy(
            e_hbm.at[pl.ds(src0, per_worker)], sidx, sem_b).wait()

        plsc.subcore_barrier()

        NSTR = 5
        QE = E_CHUNK // NSTR

        def start_gather(i, nrows, nsem):
            for q in range(NSTR):
                pltpu.async_copy(
                    x_hbm.at[sidx.at[pl.ds(i * E_CHUNK + q * QE, QE)]],
                    nrows.at[pl.ds(q * QE, QE)], nsem)

        pltpu.async_copy(e_hbm.at[pl.ds(dst0, E_CHUNK)], didx_a, dsem_a)
        start_gather(0, rows_a, sem_a)

        def step(i, rows, sem, didx, dsem, ssem,
                 nrows, nsem, ndidx, ndsem, nssem):
            @pl.when(i + 1 < n_chunks)
            def _():
                @pl.when(i >= 1)
                def _():
                    pltpu.make_async_copy(nrows, agg.at[ndidx], nssem).wait()

                pltpu.async_copy(
                    e_hbm.at[pl.ds(dst0 + (i + 1) * E_CHUNK, E_CHUNK)],
                    ndidx, ndsem)
                start_gather(i + 1, nrows, nsem)

            pltpu.make_async_copy(
                x_hbm.at[sidx.at[pl.ds(i * E_CHUNK, E_CHUNK)]],
                rows, sem).wait()
            pltpu.make_async_copy(
                e_hbm.at[pl.ds(dst0 + i * E_CHUNK, E_CHUNK)],
                didx, dsem).wait()
            pltpu.async_copy(rows, agg.at[didx], ssem, add=True)

        def body(i, carry):
            @pl.when(lax.rem(i, 2) == 0)
            def _():
                step(i, rows_a, sem_a, didx_a, dsem_a, ssem_a,
                     rows_b, sem_b, didx_b, dsem_b, ssem_b)

            @pl.when(lax.rem(i, 2) == 1)
            def _():
                step(i, rows_b, sem_b, didx_b, dsem_b, ssem_b,
                     rows_a, sem_a, didx_a, dsem_a, ssem_a)

            return carry

        lax.fori_loop(0, n_chunks, body, 0)
        pltpu.make_async_copy(rows_a, agg.at[didx_a], ssem_a).wait()
        pltpu.make_async_copy(rows_b, agg.at[didx_b], ssem_b).wait()

        plsc.subcore_barrier()

        last = N_NODES - (NS - 1) * rows_per_tile

        @pl.when(sid < NS - 1)
        def _():
            pltpu.sync_copy(agg.at[pl.ds(row0, rows_per_tile)],
                            out_hbm.at[cid, pl.ds(row0, rows_per_tile)])

        @pl.when(sid == NS - 1)
        def _():
            pltpu.sync_copy(agg.at[pl.ds((NS - 1) * rows_per_tile, last)],
                            out_hbm.at[cid, pl.ds((NS - 1) * rows_per_tile, last)])

    return k(x, eflat)


def _tc_finish(partials, x, W_rel, W_root, b2, g2, be2):
    BLK = 1000
    n_blk = N_NODES // BLK

    def body(p_ref, x_ref, wr_ref, wt_ref, b_ref, g_ref, be_ref, o_ref,
             h_scr, s_scr, s2_scr):
        ph = pl.program_id(0)
        i = pl.program_id(1)

        @pl.when(ph == 0)
        def _():
            a = p_ref[0] + p_ref[1]
            h = lax.dot_general(a, wr_ref[...], (((1,), (1,)), ((), ())),
                                preferred_element_type=jnp.float32)
            h = h + lax.dot_general(x_ref[...], wt_ref[...],
                                    (((1,), (1,)), ((), ())),
                                    preferred_element_type=jnp.float32)
            h = h + b_ref[...]
            h_scr[pl.ds(i * BLK, BLK), :] = h
            o_ref[...] = h

            @pl.when(i == 0)
            def _():
                s_scr[...] = jnp.zeros((1, D), jnp.float32)
                s2_scr[...] = jnp.zeros((1, D), jnp.float32)

            s_scr[...] += jnp.sum(h, axis=0, keepdims=True)
            s2_scr[...] += jnp.sum(h * h, axis=0, keepdims=True)

        @pl.when(ph == 1)
        def _():
            mean = s_scr[...] * (1.0 / N_NODES)
            var = s2_scr[...] * (1.0 / N_NODES) - mean * mean
            scale = g_ref[...] * lax.rsqrt(var + EPS)
            shift = be_ref[...] - mean * scale
            h = h_scr[pl.ds(i * BLK, BLK), :]
            o_ref[...] = jnp.maximum(h * scale + shift, 0.0)

    return pl.pallas_call(
        body,
        grid=(2, n_blk),
        in_specs=[
            pl.BlockSpec((NC, BLK, D), lambda ph, i: (0, (1 - ph) * i, 0)),
            pl.BlockSpec((BLK, D), lambda ph, i: ((1 - ph) * i, 0)),
            pl.BlockSpec((D, D), lambda ph, i: (0, 0)),
            pl.BlockSpec((D, D), lambda ph, i: (0, 0)),
            pl.BlockSpec((1, D), lambda ph, i: (0, 0)),
            pl.BlockSpec((1, D), lambda ph, i: (0, 0)),
            pl.BlockSpec((1, D), lambda ph, i: (0, 0)),
        ],
        out_specs=pl.BlockSpec((BLK, D), lambda ph, i: (i, 0)),
        scratch_shapes=[
            pltpu.VMEM((N_NODES, D), jnp.float32),
            pltpu.VMEM((1, D), jnp.float32),
            pltpu.VMEM((1, D), jnp.float32),
        ],
        out_shape=jax.ShapeDtypeStruct((N_NODES, D), jnp.float32),
    )(partials, x, W_rel, W_root, b2, g2, be2)


def kernel(x, edge_index, batch, W_rel, W_root, b_rel, gamma, beta):
    del batch
    ei = edge_index.astype(jnp.int32)
    E = ei.shape[1]
    per_worker = E // NW
    n_chunks = per_worker // E_CHUNK
    eflat = ei.reshape(2 * E)
    partials = _sc_segment_sum(x, eflat, n_chunks, per_worker)
    return _tc_finish(partials, x, W_rel, W_root,
                      b_rel.reshape(1, D), gamma.reshape(1, D),
                      beta.reshape(1, D))


# speedup vs baseline: 1.0833x; 1.0698x over previous
import functools

import jax
import jax.numpy as jnp
from jax import lax
from jax.experimental import pallas as pl
from jax.experimental.pallas import tpu as pltpu
from jax.experimental.pallas import tpu_sc as plsc

N_NODES = 10000
N_PAD = 10240
D = 128
EPS = 1e-5

NC = 2
NS = 16
NW = NC * NS

E_CHUNK = 128


def _sc_segment_sum(x, eflat):
    rows_per_tile = N_PAD // NS
    E = eflat.shape[0] // 2
    n_total = E // E_CHUNK
    base = n_total // NW
    rem = n_total % NW

    mesh = plsc.VectorSubcoreMesh(core_axis_name="c", subcore_axis_name="s")

    @functools.partial(
        pl.kernel,
        out_type=jax.ShapeDtypeStruct((NC, N_NODES, D), jnp.float32),
        mesh=mesh,
        scratch_types=[
            pltpu.VMEM((E_CHUNK,), jnp.int32),
            pltpu.VMEM((E_CHUNK,), jnp.int32),
            pltpu.VMEM((E_CHUNK,), jnp.int32),
            pltpu.VMEM((E_CHUNK,), jnp.int32),
            pltpu.VMEM((E_CHUNK, D), jnp.float32),
            pltpu.VMEM((E_CHUNK, D), jnp.float32),
            pltpu.VMEM_SHARED((N_PAD, D), jnp.float32),
            pltpu.SemaphoreType.DMA,
            pltpu.SemaphoreType.DMA,
            pltpu.SemaphoreType.DMA,
            pltpu.SemaphoreType.DMA,
            pltpu.SemaphoreType.DMA,
            pltpu.SemaphoreType.DMA,
            pltpu.SemaphoreType.DMA,
            pltpu.SemaphoreType.DMA,
        ],
    )
    def k(x_hbm, e_hbm, out_hbm, sbuf_a, sbuf_b, didx_a, didx_b,
          rows_a, rows_b, agg, sem_a, sem_b, dsem_a, dsem_b,
          ssem_a, ssem_b, scsem_a, scsem_b):
        cid = lax.axis_index("c")
        sid = lax.axis_index("s")
        wid = cid * NS + sid
        n_i = jnp.where(wid < rem, base + 1, base)

        def soff(i):
            return (wid + i * NW) * E_CHUNK

        def doff(i):
            return E + (wid + i * NW) * E_CHUNK

        zero16 = jnp.zeros((16,), jnp.float32)

        def zfill(r, carry):
            for j in range(D // 16):
                rows_a[r, pl.ds(j * 16, 16)] = zero16
            return carry

        lax.fori_loop(0, E_CHUNK, zfill, 0)
        row0 = sid * rows_per_tile
        for j in range(rows_per_tile // E_CHUNK):
            pltpu.async_copy(rows_a,
                             agg.at[pl.ds(row0 + j * E_CHUNK, E_CHUNK)],
                             sem_a)
        for j in range(rows_per_tile // E_CHUNK):
            pltpu.make_async_copy(
                rows_a, agg.at[pl.ds(row0 + j * E_CHUNK, E_CHUNK)],
                sem_a).wait()

        plsc.subcore_barrier()

        pltpu.async_copy(e_hbm.at[pl.ds(soff(0), E_CHUNK)], sbuf_a, scsem_a)
        pltpu.async_copy(e_hbm.at[pl.ds(soff(1), E_CHUNK)], sbuf_b, scsem_b)
        pltpu.async_copy(e_hbm.at[pl.ds(doff(0), E_CHUNK)], didx_a, dsem_a)
        pltpu.make_async_copy(
            e_hbm.at[pl.ds(soff(0), E_CHUNK)], sbuf_a, scsem_a).wait()
        pltpu.async_copy(x_hbm.at[sbuf_a], rows_a, sem_a)

        def step(i, sbuf, scsem, rows, sem, didx, dsem, ssem,
                 nsbuf, nscsem, nrows, nsem, ndidx, ndsem, nssem):
            @pl.when(i + 1 < n_i)
            def _():
                @pl.when(i >= 1)
                def _():
                    pltpu.make_async_copy(nrows, agg.at[ndidx], nssem).wait()

                pltpu.make_async_copy(
                    e_hbm.at[pl.ds(soff(i + 1), E_CHUNK)],
                    nsbuf, nscsem).wait()
                pltpu.async_copy(
                    e_hbm.at[pl.ds(doff(i + 1), E_CHUNK)], ndidx, ndsem)
                pltpu.async_copy(x_hbm.at[nsbuf], nrows, nsem)

            pltpu.make_async_copy(x_hbm.at[sbuf], rows, sem).wait()

            @pl.when(i + 2 < n_i)
            def _():
                pltpu.async_copy(
                    e_hbm.at[pl.ds(soff(i + 2), E_CHUNK)], sbuf, scsem)

            pltpu.make_async_copy(
                e_hbm.at[pl.ds(doff(i), E_CHUNK)], didx, dsem).wait()
            pltpu.async_copy(rows, agg.at[didx], ssem, add=True)

        def body(i, carry):
            @pl.when(lax.rem(i, 2) == 0)
            def _():
                step(i, sbuf_a, scsem_a, rows_a, sem_a, didx_a, dsem_a,
                     ssem_a, sbuf_b, scsem_b, rows_b, sem_b, didx_b,
                     dsem_b, ssem_b)

            @pl.when(lax.rem(i, 2) == 1)
            def _():
                step(i, sbuf_b, scsem_b, rows_b, sem_b, didx_b, dsem_b,
                     ssem_b, sbuf_a, scsem_a, rows_a, sem_a, didx_a,
                     dsem_a, ssem_a)

            return carry

        lax.fori_loop(0, n_i, body, 0)
        pltpu.make_async_copy(rows_a, agg.at[didx_a], ssem_a).wait()
        pltpu.make_async_copy(rows_b, agg.at[didx_b], ssem_b).wait()

        plsc.subcore_barrier()

        last = N_NODES - (NS - 1) * rows_per_tile

        @pl.when(sid < NS - 1)
        def _():
            pltpu.sync_copy(agg.at[pl.ds(row0, rows_per_tile)],
                            out_hbm.at[cid, pl.ds(row0, rows_per_tile)])

        @pl.when(sid == NS - 1)
        def _():
            pltpu.sync_copy(agg.at[pl.ds((NS - 1) * rows_per_tile, last)],
                            out_hbm.at[cid, pl.ds((NS - 1) * rows_per_tile, last)])

    return k(x, eflat)


def _tc_finish(partials, x, W_rel, W_root, b2, g2, be2):
    BLK = 1000
    n_blk = N_NODES // BLK

    def body(p_ref, x_ref, wr_ref, wt_ref, b_ref, g_ref, be_ref, o_ref,
             h_scr, s_scr, s2_scr):
        ph = pl.program_id(0)
        i = pl.program_id(1)

        @pl.when(ph == 0)
        def _():
            a = p_ref[0] + p_ref[1]
            h = lax.dot_general(a, wr_ref[...], (((1,), (1,)), ((), ())),
                                preferred_element_type=jnp.float32)
            h = h + lax.dot_general(x_ref[...], wt_ref[...],
                                    (((1,), (1,)), ((), ())),
                                    preferred_element_type=jnp.float32)
            h = h + b_ref[...]
            h_scr[pl.ds(i * BLK, BLK), :] = h
            o_ref[...] = h

            @pl.when(i == 0)
            def _():
                s_scr[...] = jnp.zeros((1, D), jnp.float32)
                s2_scr[...] = jnp.zeros((1, D), jnp.float32)

            s_scr[...] += jnp.sum(h, axis=0, keepdims=True)
            s2_scr[...] += jnp.sum(h * h, axis=0, keepdims=True)

        @pl.when(ph == 1)
        def _():
            mean = s_scr[...] * (1.0 / N_NODES)
            var = s2_scr[...] * (1.0 / N_NODES) - mean * mean
            scale = g_ref[...] * lax.rsqrt(var + EPS)
            shift = be_ref[...] - mean * scale
            h = h_scr[pl.ds(i * BLK, BLK), :]
            o_ref[...] = jnp.maximum(h * scale + shift, 0.0)

    return pl.pallas_call(
        body,
        grid=(2, n_blk),
        in_specs=[
            pl.BlockSpec((NC, BLK, D), lambda ph, i: (0, (1 - ph) * i, 0)),
            pl.BlockSpec((BLK, D), lambda ph, i: ((1 - ph) * i, 0)),
            pl.BlockSpec((D, D), lambda ph, i: (0, 0)),
            pl.BlockSpec((D, D), lambda ph, i: (0, 0)),
            pl.BlockSpec((1, D), lambda ph, i: (0, 0)),
            pl.BlockSpec((1, D), lambda ph, i: (0, 0)),
            pl.BlockSpec((1, D), lambda ph, i: (0, 0)),
        ],
        out_specs=pl.BlockSpec((BLK, D), lambda ph, i: (i, 0)),
        scratch_shapes=[
            pltpu.VMEM((N_NODES, D), jnp.float32),
            pltpu.VMEM((1, D), jnp.float32),
            pltpu.VMEM((1, D), jnp.float32),
        ],
        out_shape=jax.ShapeDtypeStruct((N_NODES, D), jnp.float32),
    )(partials, x, W_rel, W_root, b2, g2, be2)


def kernel(x, edge_index, batch, W_rel, W_root, b_rel, gamma, beta):
    del batch
    ei = edge_index.astype(jnp.int32)
    E = ei.shape[1]
    eflat = ei.reshape(2 * E)
    partials = _sc_segment_sum(x, eflat)
    return _tc_finish(partials, x, W_rel, W_root,
                      b_rel.reshape(1, D), gamma.reshape(1, D),
                      beta.reshape(1, D))
